# baseline (speedup 1.0000x reference)
import functools

import jax
import jax.numpy as jnp
from jax import lax
from jax.experimental import pallas as pl
from jax.experimental.pallas import tpu as pltpu
from jax.experimental.pallas import tpu_sc as plsc

F32 = jnp.float32
I32 = jnp.int32

CAP = 128
TOK_BLK = 1024


def _router_body(h_ref, w_ref, probs_ref, probs_t_ref):
    logits = jnp.dot(h_ref[...], w_ref[...], preferred_element_type=F32)
    m = jnp.max(logits, axis=1, keepdims=True)
    e = jnp.exp(logits - m)
    p = e / jnp.sum(e, axis=1, keepdims=True)
    probs_ref[...] = p
    probs_t_ref[...] = jnp.transpose(p)


def _router(h, router_W):
    n, d = h.shape
    n_e = router_W.shape[1]
    return pl.pallas_call(
        _router_body,
        grid=(n // TOK_BLK,),
        in_specs=[pl.BlockSpec((TOK_BLK, d), lambda i: (i, 0)),
                  pl.BlockSpec((d, n_e), lambda i: (0, 0))],
        out_specs=[pl.BlockSpec((TOK_BLK, n_e), lambda i: (i, 0)),
                   pl.BlockSpec((n_e, TOK_BLK), lambda i: (0, i))],
        out_shape=[jax.ShapeDtypeStruct((n, n_e), F32),
                   jax.ShapeDtypeStruct((n_e, n), F32)],
    )(h, router_W)


def _select_body(probs_t_ref, eidx_ref, posmap_ref, tokids_t_ref,
                 gates_t_ref, loss_ref, keys_ref, m_ref, eidx_row_ref,
                 tierank_ref, selrank_ref):
    n_e, n = probs_t_ref.shape
    n_blk = n // TOK_BLK
    pad_row = n_e * CAP

    p = probs_t_ref[...]
    iota_e = lax.broadcasted_iota(I32, (n_e, n), 0)
    m = jnp.max(p, axis=0, keepdims=True)
    eidx = jnp.min(jnp.where(p == m, iota_e, n_e), axis=0, keepdims=True)
    onehot = iota_e == eidx
    gbits = lax.bitcast_convert_type(m, I32)
    keys = jnp.where(onehot, gbits, 0)
    keys_ref[...] = keys
    m_ref[...] = m
    eidx_row_ref[...] = eidx

    cnt = jnp.sum(jnp.where(keys > 0, 1.0, 0.0), axis=1, keepdims=True)
    ce_sum = jnp.sum(p, axis=1, keepdims=True)
    loss_ref[...] = jnp.reshape(
        jnp.sum(cnt * ce_sum) * (n_e / (float(n) * float(n))), (1, 1))

    def bs_step(_, lohi):
        lo, hi = lohi
        mid = (lo + hi) // 2
        c = jnp.sum((keys_ref[...] > mid).astype(I32), axis=1, keepdims=True)
        take = c < CAP
        return (jnp.where(take, lo, mid), jnp.where(take, mid, hi))

    lo0 = jnp.full((n_e, 1), -1, I32)
    hi0 = jnp.full((n_e, 1), 1 << 30, I32)
    _, T = lax.fori_loop(0, 31, bs_step, (lo0, hi0))
    n_gt = jnp.sum((keys_ref[...] > T).astype(I32), axis=1, keepdims=True)
    need = (CAP - n_gt).astype(F32)
    t_pos = T > 0

    ri = lax.broadcasted_iota(I32, (TOK_BLK, TOK_BLK), 0)
    ci = lax.broadcasted_iota(I32, (TOK_BLK, TOK_BLK), 1)
    utri = (ri <= ci).astype(F32)

    def tie_of(k):
        return (k == T) & t_pos

    def prefix_pass(pred_fn, out_ref):
        def step(c, off):
            sl = pl.ds(c * TOK_BLK, TOK_BLK)
            x = pred_fn(sl)
            incl = jnp.dot(x, utri, preferred_element_type=F32)
            out_ref[:, sl] = incl - x + off
            return off + incl[:, TOK_BLK - 1:TOK_BLK]
        lax.fori_loop(0, n_blk, step, jnp.zeros((n_e, 1), F32))

    prefix_pass(lambda sl: tie_of(keys_ref[:, sl]).astype(F32), tierank_ref)

    def sel_of(sl):
        k = keys_ref[:, sl]
        return (k > T) | (tie_of(k) & (tierank_ref[:, sl] < need))

    prefix_pass(lambda sl: sel_of(sl).astype(F32), selrank_ref)

    iota_cap = lax.broadcasted_iota(I32, (CAP, 1), 0)
    iota_lane = lax.broadcasted_iota(I32, (1, TOK_BLK), 1)
    iota_e_row = lax.broadcasted_iota(I32, (1, n_e), 1)

    def chunk_step(c, accs):
        ids_acc, g_acc = accs
        sl = pl.ds(c * TOK_BLK, TOK_BLK)
        sel = sel_of(sl)
        srank = selrank_ref[:, sl]
        rank_own = jnp.sum(jnp.where(sel, srank, 0.0), axis=0, keepdims=True)
        sel_tok = jnp.any(sel, axis=0, keepdims=True)
        e_own = eidx_row_ref[:, sl]
        posmap_ref[pl.ds(c, 1), :] = jnp.where(
            sel_tok, e_own * CAP + rank_own.astype(I32), pad_row)
        eidx_ref[pl.ds(c, 1), :] = e_own
        tokid = (c * TOK_BLK + iota_lane).astype(F32)
        m_c = m_ref[:, sl]
        srank_i = srank.astype(I32)
        for e in range(n_e):
            cmp = (srank_i[e:e + 1, :] == iota_cap) & sel[e:e + 1, :]
            lane_e = (iota_e_row == e).astype(F32)
            ids_acc = ids_acc + lane_e * jnp.sum(
                jnp.where(cmp, tokid, 0.0), axis=1, keepdims=True)
            g_acc = g_acc + lane_e * jnp.sum(
                jnp.where(cmp, m_c, 0.0), axis=1, keepdims=True)
        return ids_acc, g_acc

    z = jnp.zeros((CAP, n_e), F32)
    ids_acc, g_acc = lax.fori_loop(0, n_blk, chunk_step, (z, z))
    tokids_t_ref[...] = ids_acc.astype(I32)
    gates_t_ref[...] = g_acc


def _select(probs_t):
    n_e, n = probs_t.shape
    n_blk = n // TOK_BLK
    return pl.pallas_call(
        _select_body,
        out_shape=[jax.ShapeDtypeStruct((n_blk, TOK_BLK), I32),
                   jax.ShapeDtypeStruct((n_blk, TOK_BLK), I32),
                   jax.ShapeDtypeStruct((CAP, n_e), I32),
                   jax.ShapeDtypeStruct((CAP, n_e), F32),
                   jax.ShapeDtypeStruct((1, 1), F32)],
        scratch_shapes=[pltpu.VMEM((n_e, n), I32),
                        pltpu.VMEM((1, n), F32),
                        pltpu.VMEM((1, n), I32),
                        pltpu.VMEM((n_e, n), F32),
                        pltpu.VMEM((n_e, n), F32)],
    )(probs_t)


def _ffn_body(x_ref, w1_ref, b1_ref, w2_ref, b2_ref, g_ref, y_ref):
    h1 = jnp.dot(x_ref[...], w1_ref[0], preferred_element_type=F32) + b1_ref[0]
    h1 = jnp.maximum(h1, 0.0)
    y = jnp.dot(h1, w2_ref[0], preferred_element_type=F32) + b2_ref[0]
    y_ref[...] = y * g_ref[0]


def _ffn(x, W1, b1, W2, b2, gates):
    n_e, d, d_ff = W1.shape
    return pl.pallas_call(
        _ffn_body,
        grid=(n_e,),
        in_specs=[pl.BlockSpec((CAP, d), lambda e: (e, 0)),
                  pl.BlockSpec((1, d, d_ff), lambda e: (e, 0, 0)),
                  pl.BlockSpec((1, 1, d_ff), lambda e: (e, 0, 0)),
                  pl.BlockSpec((1, d_ff, d), lambda e: (e, 0, 0)),
                  pl.BlockSpec((1, 1, d), lambda e: (e, 0, 0)),
                  pl.BlockSpec((1, CAP, 1), lambda e: (e, 0, 0))],
        out_specs=pl.BlockSpec((CAP, d), lambda e: (e, 0)),
        out_shape=jax.ShapeDtypeStruct((n_e * CAP, d), F32),
    )(x, W1, b1, W2, b2, gates)


def _sc_gather(table, ids, chunk):
    n_rows = ids.shape[0]
    d = table.shape[1]
    info = plsc.get_sparse_core_info()
    nw = info.num_cores * info.num_subcores
    per = n_rows // nw
    n_ch = per // chunk
    mesh = plsc.VectorSubcoreMesh(core_axis_name="c", subcore_axis_name="s")

    @functools.partial(
        pl.kernel, mesh=mesh,
        out_type=jax.ShapeDtypeStruct((n_rows, d), F32),
        scratch_types=[pltpu.VMEM((chunk,), I32),
                       pltpu.VMEM((chunk, d), F32),
                       pltpu.SemaphoreType.DMA])
    def gk(tab, idx_hbm, out, idx_v, rows_v, sem):
        wid = lax.axis_index("s") * info.num_cores + lax.axis_index("c")
        base = wid * per
        for k in range(n_ch):
            sl = pl.ds(base + k * chunk, chunk)
            pltpu.sync_copy(idx_hbm.at[sl], idx_v)
            pltpu.async_copy(tab.at[idx_v], rows_v, sem).wait()
            pltpu.sync_copy(rows_v, out.at[sl])

    return gk(table, ids)


def kernel(hidden_states, router_W, W1, b1, W2, b2):
    bq, sq, d = hidden_states.shape
    n = bq * sq
    n_e, _, d_ff = W1.shape
    h = hidden_states.reshape(n, d)

    probs, probs_t = _router(h, router_W)
    eidx, posmap, tokids_t, gates_t, loss = _select(probs_t)

    x = _sc_gather(h, tokids_t.T.reshape(-1), 32)
    y = _ffn(x, W1, b1.reshape(n_e, 1, d_ff), W2, b2.reshape(n_e, 1, d),
             gates_t.T.reshape(n_e, CAP, 1))
    y_pad = jnp.concatenate([y, jnp.zeros((8, d), F32)], axis=0)
    out = _sc_gather(y_pad, posmap.reshape(-1), 128)

    return (out.reshape(bq, sq, d), loss.reshape(()), probs,
            eidx.reshape(n))


# speedup vs baseline: 3.8029x; 3.8029x over previous
import functools

import jax
import jax.numpy as jnp
from jax import lax
from jax.experimental import pallas as pl
from jax.experimental.pallas import tpu as pltpu
from jax.experimental.pallas import tpu_sc as plsc

F32 = jnp.float32
I32 = jnp.int32

CAP = 128
TOK_BLK = 1024


def _router_body(h_ref, w_ref, probs_ref, probs_t_ref):
    logits = jnp.dot(h_ref[...], w_ref[...], preferred_element_type=F32)
    m = jnp.max(logits, axis=1, keepdims=True)
    e = jnp.exp(logits - m)
    p = e / jnp.sum(e, axis=1, keepdims=True)
    probs_ref[...] = p
    probs_t_ref[...] = jnp.transpose(p)


def _router(h, router_W):
    n, d = h.shape
    n_e = router_W.shape[1]
    return pl.pallas_call(
        _router_body,
        grid=(n // TOK_BLK,),
        in_specs=[pl.BlockSpec((TOK_BLK, d), lambda i: (i, 0)),
                  pl.BlockSpec((d, n_e), lambda i: (0, 0))],
        out_specs=[pl.BlockSpec((TOK_BLK, n_e), lambda i: (i, 0)),
                   pl.BlockSpec((n_e, TOK_BLK), lambda i: (0, i))],
        out_shape=[jax.ShapeDtypeStruct((n, n_e), F32),
                   jax.ShapeDtypeStruct((n_e, n), F32)],
    )(h, router_W)


def _select_body(probs_t_ref, eidx_ref, tokids_t_ref,
                 gates_t_ref, loss_ref, keys_ref, m_ref, eidx_row_ref,
                 tierank_ref, selrank_ref, unselrank_ref):
    n_e, n = probs_t_ref.shape
    n_blk = n // TOK_BLK

    p = probs_t_ref[...]
    iota_e = lax.broadcasted_iota(I32, (n_e, n), 0)
    m = jnp.max(p, axis=0, keepdims=True)
    eidx = jnp.min(jnp.where(p == m, iota_e, n_e), axis=0, keepdims=True)
    onehot = iota_e == eidx
    gbits = lax.bitcast_convert_type(m, I32)
    keys = jnp.where(onehot, gbits, 0)
    keys_ref[...] = keys
    m_ref[...] = m
    eidx_row_ref[...] = eidx

    cnt = jnp.sum(jnp.where(keys > 0, 1.0, 0.0), axis=1, keepdims=True)
    ce_sum = jnp.sum(p, axis=1, keepdims=True)
    loss_ref[...] = jnp.reshape(
        jnp.sum(cnt * ce_sum) * (n_e / (float(n) * float(n))), (1, 1))

    def bs_step(_, lohi):
        lo, hi = lohi
        mid = (lo + hi) // 2
        c = jnp.sum((keys_ref[...] > mid).astype(I32), axis=1, keepdims=True)
        take = c < CAP
        return (jnp.where(take, lo, mid), jnp.where(take, mid, hi))

    lo0 = jnp.full((n_e, 1), -1, I32)
    hi0 = jnp.full((n_e, 1), 1 << 30, I32)
    _, T = lax.fori_loop(0, 31, bs_step, (lo0, hi0))
    n_gt = jnp.sum((keys_ref[...] > T).astype(I32), axis=1, keepdims=True)
    need = (CAP - n_gt).astype(F32)
    t_pos = T > 0

    ri = lax.broadcasted_iota(I32, (TOK_BLK, TOK_BLK), 0)
    ci = lax.broadcasted_iota(I32, (TOK_BLK, TOK_BLK), 1)
    utri = (ri <= ci).astype(F32)

    def tie_of(k):
        return (k == T) & t_pos

    def prefix_pass(pred_fn, out_ref, rows):
        def step(c, off):
            sl = pl.ds(c * TOK_BLK, TOK_BLK)
            x = pred_fn(sl)
            incl = jnp.dot(x, utri, preferred_element_type=F32)
            out_ref[:, sl] = incl - x + off
            return off + incl[:, TOK_BLK - 1:TOK_BLK]
        lax.fori_loop(0, n_blk, step, jnp.zeros((rows, 1), F32))

    prefix_pass(lambda sl: tie_of(keys_ref[:, sl]).astype(F32), tierank_ref,
                n_e)

    def sel_of(sl):
        k = keys_ref[:, sl]
        return (k > T) | (tie_of(k) & (tierank_ref[:, sl] < need))

    prefix_pass(lambda sl: sel_of(sl).astype(F32), selrank_ref, n_e)
    prefix_pass(
        lambda sl: 1.0 - jnp.any(sel_of(sl), axis=0, keepdims=True).astype(F32),
        unselrank_ref, 1)

    n_tie = jnp.sum(tie_of(keys_ref[...]).astype(I32), axis=1, keepdims=True)
    n_sel = (n_gt + jnp.minimum(CAP - n_gt, n_tie)).astype(F32)
    ser = lax.broadcasted_iota(I32, (n_e, n_e), 0)
    sec = lax.broadcasted_iota(I32, (n_e, n_e), 1)
    stri = (sec < ser).astype(F32)
    cum_empty = jnp.dot(stri, CAP - n_sel, preferred_element_type=F32)

    iota_cap = lax.broadcasted_iota(I32, (CAP, 1), 0)
    iota_lane = lax.broadcasted_iota(I32, (1, TOK_BLK), 1)
    iota_e_row = lax.broadcasted_iota(I32, (1, n_e), 1)

    n_sel_i = n_sel.astype(I32)
    cum_empty_i = cum_empty.astype(I32)

    def chunk_step(c, accs):
        ids_acc, g_acc = accs
        sl = pl.ds(c * TOK_BLK, TOK_BLK)
        sel = sel_of(sl)
        srank = selrank_ref[:, sl]
        sel_tok = jnp.any(sel, axis=0, keepdims=True)
        eidx_ref[pl.ds(c, 1), :] = eidx_row_ref[:, sl]
        tokid = (c * TOK_BLK + iota_lane).astype(F32)
        m_c = m_ref[:, sl]
        srank_i = srank.astype(I32)
        urank_i = unselrank_ref[:, sl].astype(I32)
        for e in range(n_e):
            cmp = (srank_i[e:e + 1, :] == iota_cap) & sel[e:e + 1, :]
            fill = ((urank_i == iota_cap - n_sel_i[e:e + 1, :]
                     + cum_empty_i[e:e + 1, :])
                    & jnp.logical_not(sel_tok)
                    & (iota_cap >= n_sel_i[e:e + 1, :]))
            lane_e = (iota_e_row == e).astype(F32)
            ids_acc = ids_acc + lane_e * jnp.sum(
                jnp.where(cmp | fill, tokid, 0.0), axis=1, keepdims=True)
            g_acc = g_acc + lane_e * jnp.sum(
                jnp.where(cmp, m_c, 0.0), axis=1, keepdims=True)
        return ids_acc, g_acc

    z = jnp.zeros((CAP, n_e), F32)
    ids_acc, g_acc = lax.fori_loop(0, n_blk, chunk_step, (z, z))
    tokids_t_ref[...] = ids_acc.astype(I32)
    gates_t_ref[...] = g_acc


def _select(probs_t):
    n_e, n = probs_t.shape
    n_blk = n // TOK_BLK
    return pl.pallas_call(
        _select_body,
        out_shape=[jax.ShapeDtypeStruct((n_blk, TOK_BLK), I32),
                   jax.ShapeDtypeStruct((CAP, n_e), I32),
                   jax.ShapeDtypeStruct((CAP, n_e), F32),
                   jax.ShapeDtypeStruct((1, 1), F32)],
        scratch_shapes=[pltpu.VMEM((n_e, n), I32),
                        pltpu.VMEM((1, n), F32),
                        pltpu.VMEM((1, n), I32),
                        pltpu.VMEM((n_e, n), F32),
                        pltpu.VMEM((n_e, n), F32),
                        pltpu.VMEM((1, n), F32)],
    )(probs_t)


def _ffn_body(x_ref, w1_ref, b1_ref, w2_ref, b2_ref, g_ref, y_ref):
    h1 = jnp.dot(x_ref[...], w1_ref[0], preferred_element_type=F32) + b1_ref[0]
    h1 = jnp.maximum(h1, 0.0)
    y = jnp.dot(h1, w2_ref[0], preferred_element_type=F32) + b2_ref[0]
    y_ref[...] = y * g_ref[0]


def _ffn(x, W1, b1, W2, b2, gates):
    n_e, d, d_ff = W1.shape
    return pl.pallas_call(
        _ffn_body,
        grid=(n_e,),
        in_specs=[pl.BlockSpec((CAP, d), lambda e: (e, 0)),
                  pl.BlockSpec((1, d, d_ff), lambda e: (e, 0, 0)),
                  pl.BlockSpec((1, 1, d_ff), lambda e: (e, 0, 0)),
                  pl.BlockSpec((1, d_ff, d), lambda e: (e, 0, 0)),
                  pl.BlockSpec((1, 1, d), lambda e: (e, 0, 0)),
                  pl.BlockSpec((1, CAP, 1), lambda e: (e, 0, 0))],
        out_specs=pl.BlockSpec((CAP, d), lambda e: (e, 0)),
        out_shape=jax.ShapeDtypeStruct((n_e * CAP, d), F32),
    )(x, W1, b1, W2, b2, gates)


def _sc_gather(table, ids, chunk):
    n_rows = ids.shape[0]
    d = table.shape[1]
    info = plsc.get_sparse_core_info()
    nw = info.num_cores * info.num_subcores
    per = n_rows // nw
    n_ch = per // chunk
    mesh = plsc.VectorSubcoreMesh(core_axis_name="c", subcore_axis_name="s")

    @functools.partial(
        pl.kernel, mesh=mesh,
        out_type=jax.ShapeDtypeStruct((n_rows, d), F32),
        scratch_types=[pltpu.VMEM((chunk,), I32),
                       pltpu.VMEM((chunk, d), F32),
                       pltpu.SemaphoreType.DMA])
    def gk(tab, idx_hbm, out, idx_v, rows_v, sem):
        wid = lax.axis_index("s") * info.num_cores + lax.axis_index("c")
        base = wid * per
        for k in range(n_ch):
            sl = pl.ds(base + k * chunk, chunk)
            pltpu.sync_copy(idx_hbm.at[sl], idx_v)
            pltpu.async_copy(tab.at[idx_v], rows_v, sem).wait()
            pltpu.sync_copy(rows_v, out.at[sl])

    return gk(table, ids)


def _sc_zero_scatter(y, ids, zeros_blk, n):
    n_slots, d = y.shape
    info = plsc.get_sparse_core_info()
    ns = info.num_subcores
    rows_per_w = n // ns
    zrows = zeros_blk.shape[0]
    nz = rows_per_w // zrows
    sl_per_w = n_slots // ns
    mesh = plsc.VectorSubcoreMesh(core_axis_name="c", subcore_axis_name="s",
                                  num_cores=1)

    @functools.partial(
        pl.kernel, mesh=mesh,
        out_type=jax.ShapeDtypeStruct((n, d), F32),
        scratch_types=[pltpu.VMEM((zrows, d), F32),
                       pltpu.VMEM((sl_per_w,), I32),
                       pltpu.VMEM((sl_per_w, d), F32),
                       pltpu.SemaphoreType.DMA])
    def zk(y_hbm, ids_hbm, z_hbm, out, zbuf, idx_v, rows_v, sem):
        wid = lax.axis_index("s")
        base = wid * rows_per_w
        pltpu.sync_copy(z_hbm, zbuf)
        for k in range(nz):
            pltpu.sync_copy(zbuf, out.at[pl.ds(base + k * zrows, zrows)])
        plsc.subcore_barrier()
        sbase = wid * sl_per_w
        pltpu.sync_copy(ids_hbm.at[pl.ds(sbase, sl_per_w)], idx_v)
        pltpu.sync_copy(y_hbm.at[pl.ds(sbase, sl_per_w)], rows_v)
        pltpu.async_copy(rows_v, out.at[idx_v], sem).wait()

    return zk(y, ids, zeros_blk)


def kernel(hidden_states, router_W, W1, b1, W2, b2):
    bq, sq, d = hidden_states.shape
    n = bq * sq
    n_e, _, d_ff = W1.shape
    h = hidden_states.reshape(n, d)

    probs, probs_t = _router(h, router_W)
    eidx, tokids_t, gates_t, loss = _select(probs_t)

    ids = tokids_t.T.reshape(-1)
    x = _sc_gather(h, ids, 32)
    y = _ffn(x, W1, b1.reshape(n_e, 1, d_ff), W2, b2.reshape(n_e, 1, d),
             gates_t.T.reshape(n_e, CAP, 1))
    out = _sc_zero_scatter(y, ids, jnp.zeros((64, d), F32), n)

    return (out.reshape(bq, sq, d), loss.reshape(()), probs,
            eidx.reshape(n))
